# baseline (speedup 1.0000x reference)
import functools

import jax
import jax.numpy as jnp
from jax import lax
from jax.experimental import pallas as pl
from jax.experimental.pallas import tpu as pltpu
from jax.experimental.pallas import tpu_sc as plsc

VOCAB = 1000000
DIM = 64
BATCH = 16384
NEG = 5

NC = 2
NS = 16
L = 16
NW = NC * NS
B_PER_W = BATCH // NW
CHUNK = 128
NCHUNK = B_PER_W // CHUNK

NTILE = VOCAB // 128
TPW = -(-NTILE // NW)
VPAD = NTILE * 128 + 128


def _repack(embT, stub):
    mesh = plsc.VectorSubcoreMesh(core_axis_name="c", subcore_axis_name="s")

    @functools.partial(
        pl.kernel,
        out_type=jax.ShapeDtypeStruct((VPAD, 128), jnp.float32),
        mesh=mesh,
        compiler_params=pltpu.CompilerParams(
            needs_layout_passes=False, use_tc_tiling_on_sc=True),
        scratch_types=[
            pltpu.VMEM((DIM, 128), jnp.float32),
            pltpu.VMEM((128, 128), jnp.float32),
            pltpu.SemaphoreType.DMA,
        ],
    )
    def k(embT_hbm, stub_hbm, out_hbm, tbuf, obuf, sem):
        wid = lax.axis_index("s") * NC + lax.axis_index("c")
        iota = lax.iota(jnp.int32, L)
        t0 = wid * TPW
        ntiles = jnp.minimum(TPW, jnp.maximum(NTILE - t0, 0))

        def tile_body(tt, carry):
            t = t0 + tt
            pltpu.async_copy(
                embT_hbm.at[:, pl.ds(t * 128, 128)], tbuf, sem).wait()
            for cg in range(8):
                cols = cg * L + iota
                for d in range(DIM):
                    v = tbuf[d, pl.ds(cg * L, L)]
                    plsc.store_scatter(obuf, [cols, jnp.full((L,), d,
                                                             jnp.int32)], v)
            pltpu.sync_copy(obuf, out_hbm.at[pl.ds(t * 128, 128)])
            return carry

        lax.fori_loop(0, ntiles, tile_body, 0)

        @pl.when(wid == NW - 1)
        def _():
            pltpu.async_copy(stub_hbm, tbuf, sem).wait()
            for cg in range(8):
                cols = cg * L + iota
                for d in range(DIM):
                    v = tbuf[d, pl.ds(cg * L, L)]
                    plsc.store_scatter(obuf, [cols, jnp.full((L,), d,
                                                             jnp.int32)], v)
            pltpu.sync_copy(obuf, out_hbm.at[pl.ds(NTILE * 128, 128)])

    return k(embT, stub)


def _sc_scores(target_words, context_words, neg_flat, ptab_i, ptab_o):
    mesh = plsc.VectorSubcoreMesh(core_axis_name="c", subcore_axis_name="s")

    @functools.partial(
        pl.kernel,
        out_type=[
            jax.ShapeDtypeStruct((BATCH,), jnp.float32),
            jax.ShapeDtypeStruct((BATCH * NEG,), jnp.float32),
        ],
        mesh=mesh,
        compiler_params=pltpu.CompilerParams(
            needs_layout_passes=False, use_tc_tiling_on_sc=True),
        scratch_types=[
            pltpu.VMEM((CHUNK,), jnp.int32),
            pltpu.VMEM((CHUNK,), jnp.int32),
            pltpu.VMEM((NEG * CHUNK,), jnp.int32),
            pltpu.VMEM((CHUNK, 128), jnp.float32),
            pltpu.VMEM((CHUNK, 128), jnp.float32),
            pltpu.VMEM((NEG * CHUNK, 128), jnp.float32),
            pltpu.VMEM((CHUNK,), jnp.float32),
            pltpu.VMEM((NEG * CHUNK,), jnp.float32),
            pltpu.SemaphoreType.DMA,
        ],
    )
    def k(tw_hbm, cw_hbm, nw_hbm, iemb_hbm, oemb_hbm, pos_hbm, negout_hbm,
          t_idx, c_idx, n_idx, t_rows, c_rows, n_rows, pos_v, neg_v, sem):
        wid = lax.axis_index("s") * NC + lax.axis_index("c")
        iota = lax.iota(jnp.int32, L)
        for ch in range(NCHUNK):
            base = wid * B_PER_W + ch * CHUNK
            stage = [
                pltpu.async_copy(tw_hbm.at[pl.ds(base, CHUNK)], t_idx, sem),
                pltpu.async_copy(cw_hbm.at[pl.ds(base, CHUNK)], c_idx, sem),
                pltpu.async_copy(
                    nw_hbm.at[pl.ds(base * NEG, NEG * CHUNK)], n_idx, sem),
            ]
            for cp in stage:
                cp.wait()
            cps = [
                pltpu.async_copy(iemb_hbm.at[t_idx], t_rows, sem),
                pltpu.async_copy(oemb_hbm.at[c_idx], c_rows, sem),
            ]
            for s in range(NEG):
                cps.append(pltpu.async_copy(
                    oemb_hbm.at[n_idx.at[pl.ds(s * CHUNK, CHUNK)]],
                    n_rows.at[pl.ds(s * CHUNK, CHUNK)], sem))
            for cp in cps:
                cp.wait()

            for blk in range(CHUNK // L):
                rows = blk * L + iota
                n_rowidx = [rows * NEG + kk for kk in range(NEG)]
                zero = jnp.zeros((L,), jnp.float32)

                def body(dd, carry, rows=rows, n_rowidx=n_rowidx):
                    accp, accn = carry[0], list(carry[1:])
                    col = jnp.full((L,), dd, jnp.int32)
                    tv = plsc.load_gather(t_rows, [rows, col])
                    cv = plsc.load_gather(c_rows, [rows, col])
                    accp = accp + tv * cv
                    for kk in range(NEG):
                        nv = plsc.load_gather(n_rows, [n_rowidx[kk], col])
                        accn[kk] = accn[kk] + tv * nv
                    return (accp, *accn)

                accs = lax.fori_loop(0, DIM, body, (zero,) * (1 + NEG))
                pos_v[pl.ds(blk * L, L)] = accs[0]
                for kk in range(NEG):
                    plsc.store_scatter(neg_v, [n_rowidx[kk]], accs[1 + kk])

            pltpu.sync_copy(pos_v, pos_hbm.at[pl.ds(base, CHUNK)])
            pltpu.sync_copy(neg_v, negout_hbm.at[pl.ds(base * NEG, NEG * CHUNK)])

    return k(target_words, context_words, neg_flat, ptab_i, ptab_o)


def _tc_loss(pos_scores, neg_scores):
    def body(p_ref, n_ref, pos_out, neg_out):
        p = p_ref[...]
        n = n_ref[...]

        def neg_logsig(x):
            return jnp.log(1.0 + jnp.exp(-jnp.abs(x))) - jnp.minimum(x, 0.0)

        pos_out[0, 0] = jnp.mean(neg_logsig(p))
        neg_out[0, 0] = jnp.mean(neg_logsig(-n))

    o1, o2 = pl.pallas_call(
        body,
        out_shape=[jax.ShapeDtypeStruct((1, 1), jnp.float32)] * 2,
        out_specs=[pl.BlockSpec(memory_space=pltpu.SMEM)] * 2,
    )(pos_scores.reshape(BATCH // 128, 128),
      neg_scores.reshape(BATCH * NEG // 128, 128))
    return o1[0, 0], o2[0, 0]


def kernel(target_words, context_words, negative_words, input_emb, output_emb):
    iT, oT = input_emb.T, output_emb.T
    stub_i = jnp.pad(iT[:, NTILE * 128:], ((0, 0), (0, 64)))
    stub_o = jnp.pad(oT[:, NTILE * 128:], ((0, 0), (0, 64)))
    ptab_i = _repack(iT, stub_i)
    ptab_o = _repack(oT, stub_o)
    pos_s, neg_s = _sc_scores(target_words, context_words,
                              negative_words.reshape(BATCH * NEG),
                              ptab_i, ptab_o)
    return _tc_loss(pos_s, neg_s)


# speedup vs baseline: 3.1948x; 3.1948x over previous
import functools

import jax
import jax.numpy as jnp
from jax import lax
from jax.experimental import pallas as pl
from jax.experimental.pallas import tpu as pltpu
from jax.experimental.pallas import tpu_sc as plsc

VOCAB = 1000000
DIM = 64
BATCH = 16384
NEG = 5

NC = 2
NS = 16
L = 16
NW = NC * NS
B_PER_W = BATCH // NW
CHUNK = 128
NCHUNK = B_PER_W // CHUNK

RW = 2048
NRBLK = -(-VOCAB // RW)
PROWS = NRBLK * (RW // 2)


def _repack(embT):
    def body(in_ref, out_ref):
        a = in_ref[...]
        out_ref[...] = jnp.concatenate(
            [a[:, :RW // 2].T, a[:, RW // 2:].T], axis=1)

    return pl.pallas_call(
        body,
        grid=(NRBLK,),
        in_specs=[pl.BlockSpec((DIM, RW), lambda g: (0, g))],
        out_specs=pl.BlockSpec((RW // 2, 128), lambda g: (g, 0)),
        out_shape=jax.ShapeDtypeStruct((PROWS, 128), jnp.float32),
    )(embT)


def _sc_scores(t_div, t_off, c_div, c_off, n_div, n_off, ptab_i, ptab_o):
    mesh = plsc.VectorSubcoreMesh(core_axis_name="c", subcore_axis_name="s")

    @functools.partial(
        pl.kernel,
        out_type=[
            jax.ShapeDtypeStruct((BATCH,), jnp.float32),
            jax.ShapeDtypeStruct((BATCH * NEG,), jnp.float32),
        ],
        mesh=mesh,
        compiler_params=pltpu.CompilerParams(
            needs_layout_passes=False, use_tc_tiling_on_sc=True),
        scratch_types=[
            pltpu.VMEM((CHUNK,), jnp.int32),
            pltpu.VMEM((CHUNK,), jnp.int32),
            pltpu.VMEM((CHUNK,), jnp.int32),
            pltpu.VMEM((CHUNK,), jnp.int32),
            pltpu.VMEM((NEG * CHUNK,), jnp.int32),
            pltpu.VMEM((NEG * CHUNK,), jnp.int32),
            pltpu.VMEM((CHUNK, 128), jnp.float32),
            pltpu.VMEM((CHUNK, 128), jnp.float32),
            pltpu.VMEM((NEG * CHUNK, 128), jnp.float32),
            pltpu.VMEM((CHUNK,), jnp.float32),
            pltpu.VMEM((NEG * CHUNK,), jnp.float32),
            pltpu.SemaphoreType.DMA,
        ],
    )
    def k(td_hbm, to_hbm, cd_hbm, co_hbm, nd_hbm, no_hbm, iemb_hbm, oemb_hbm,
          pos_hbm, negout_hbm,
          t_idx, t_po, c_idx, c_po, n_idx, n_po,
          t_rows, c_rows, n_rows, pos_v, neg_v, sem):
        wid = lax.axis_index("s") * NC + lax.axis_index("c")
        iota = lax.iota(jnp.int32, L)
        for ch in range(NCHUNK):
            base = wid * B_PER_W + ch * CHUNK
            stage = [
                pltpu.async_copy(td_hbm.at[pl.ds(base, CHUNK)], t_idx, sem),
                pltpu.async_copy(to_hbm.at[pl.ds(base, CHUNK)], t_po, sem),
                pltpu.async_copy(cd_hbm.at[pl.ds(base, CHUNK)], c_idx, sem),
                pltpu.async_copy(co_hbm.at[pl.ds(base, CHUNK)], c_po, sem),
                pltpu.async_copy(
                    nd_hbm.at[pl.ds(base * NEG, NEG * CHUNK)], n_idx, sem),
                pltpu.async_copy(
                    no_hbm.at[pl.ds(base * NEG, NEG * CHUNK)], n_po, sem),
            ]
            for cp in stage:
                cp.wait()
            cps = [
                pltpu.async_copy(iemb_hbm.at[t_idx], t_rows, sem),
                pltpu.async_copy(oemb_hbm.at[c_idx], c_rows, sem),
            ]
            for s in range(NEG):
                cps.append(pltpu.async_copy(
                    oemb_hbm.at[n_idx.at[pl.ds(s * CHUNK, CHUNK)]],
                    n_rows.at[pl.ds(s * CHUNK, CHUNK)], sem))
            for cp in cps:
                cp.wait()

            for blk in range(CHUNK // L):
                rows = blk * L + iota
                n_rowidx = [rows * NEG + kk for kk in range(NEG)]
                t_col0 = t_po[pl.ds(blk * L, L)]
                c_col0 = c_po[pl.ds(blk * L, L)]
                n_col0 = [plsc.load_gather(n_po, [n_rowidx[kk]])
                          for kk in range(NEG)]
                zero = jnp.zeros((L,), jnp.float32)

                def body(dd, carry, rows=rows, n_rowidx=n_rowidx,
                         t_col0=t_col0, c_col0=c_col0, n_col0=n_col0):
                    accp, accn = carry[0], list(carry[1:])
                    tv = plsc.load_gather(t_rows, [rows, t_col0 + dd])
                    cv = plsc.load_gather(c_rows, [rows, c_col0 + dd])
                    accp = accp + tv * cv
                    for kk in range(NEG):
                        nv = plsc.load_gather(
                            n_rows, [n_rowidx[kk], n_col0[kk] + dd])
                        accn[kk] = accn[kk] + tv * nv
                    return (accp, *accn)

                accs = lax.fori_loop(0, DIM, body, (zero,) * (1 + NEG))
                pos_v[pl.ds(blk * L, L)] = accs[0]
                for kk in range(NEG):
                    plsc.store_scatter(neg_v, [n_rowidx[kk]], accs[1 + kk])

            pltpu.sync_copy(pos_v, pos_hbm.at[pl.ds(base, CHUNK)])
            pltpu.sync_copy(neg_v, negout_hbm.at[pl.ds(base * NEG, NEG * CHUNK)])

    return k(t_div, t_off, c_div, c_off, n_div, n_off, ptab_i, ptab_o)


def _tc_loss(pos_scores, neg_scores):
    def body(p_ref, n_ref, pos_out, neg_out):
        p = p_ref[...]
        n = n_ref[...]

        def neg_logsig(x):
            return jnp.log(1.0 + jnp.exp(-jnp.abs(x))) - jnp.minimum(x, 0.0)

        pos_out[0, 0] = jnp.mean(neg_logsig(p))
        neg_out[0, 0] = jnp.mean(neg_logsig(-n))

    o1, o2 = pl.pallas_call(
        body,
        out_shape=[jax.ShapeDtypeStruct((1, 1), jnp.float32)] * 2,
        out_specs=[pl.BlockSpec(memory_space=pltpu.SMEM)] * 2,
    )(pos_scores.reshape(BATCH // 128, 128),
      neg_scores.reshape(BATCH * NEG // 128, 128))
    return o1[0, 0], o2[0, 0]


def kernel(target_words, context_words, negative_words, input_emb, output_emb):
    ptab_i = _repack(input_emb.T)
    ptab_o = _repack(output_emb.T)
    nf = negative_words.reshape(BATCH * NEG)

    def split(ix):
        blk = jnp.right_shift(ix, 11)
        loc = jnp.bitwise_and(ix, 2047)
        div = jnp.left_shift(blk, 10) + jnp.bitwise_and(loc, 1023)
        off = jnp.left_shift(jnp.right_shift(loc, 10), 6)
        return div, off

    t_div, t_off = split(target_words)
    c_div, c_off = split(context_words)
    n_div, n_off = split(nf)
    pos_s, neg_s = _sc_scores(t_div, t_off, c_div, c_off, n_div, n_off,
                              ptab_i, ptab_o)
    return _tc_loss(pos_s, neg_s)


# speedup vs baseline: 4.8870x; 1.5297x over previous
import functools

import jax
import jax.numpy as jnp
from jax import lax
from jax.experimental import pallas as pl
from jax.experimental.pallas import tpu as pltpu
from jax.experimental.pallas import tpu_sc as plsc

VOCAB = 1000000
DIM = 64
BATCH = 16384
NEG = 5

NC = 2
NS = 16
L = 16
NW = NC * NS
B_PER_W = BATCH // NW
CHUNK = 128
NCHUNK = B_PER_W // CHUNK

RW = 8192
NRBLK = -(-VOCAB // RW)
PROWS = NRBLK * (RW // 2)


def _repack(embT):
    def body(in_ref, out_ref):
        a = in_ref[...]
        out_ref[...] = jnp.concatenate(
            [a[:, :RW // 2].T, a[:, RW // 2:].T], axis=1)

    return pl.pallas_call(
        body,
        grid=(NRBLK,),
        in_specs=[pl.BlockSpec((DIM, RW), lambda g: (0, g))],
        out_specs=pl.BlockSpec((RW // 2, 128), lambda g: (g, 0)),
        out_shape=jax.ShapeDtypeStruct((PROWS, 128), jnp.float32),
    )(embT)


def _sc_scores(t_div, t_off, c_div, c_off, n_div, n_off, ptab_i, ptab_o):
    mesh = plsc.VectorSubcoreMesh(core_axis_name="c", subcore_axis_name="s")

    @functools.partial(
        pl.kernel,
        out_type=[
            jax.ShapeDtypeStruct((BATCH,), jnp.float32),
            jax.ShapeDtypeStruct((BATCH * NEG,), jnp.float32),
        ],
        mesh=mesh,
        compiler_params=pltpu.CompilerParams(
            needs_layout_passes=False, use_tc_tiling_on_sc=True),
        scratch_types=[
            pltpu.VMEM((CHUNK,), jnp.int32),
            pltpu.VMEM((CHUNK,), jnp.int32),
            pltpu.VMEM((CHUNK,), jnp.int32),
            pltpu.VMEM((CHUNK,), jnp.int32),
            pltpu.VMEM((NEG * CHUNK,), jnp.int32),
            pltpu.VMEM((NEG * CHUNK,), jnp.int32),
            pltpu.VMEM((CHUNK, 128), jnp.float32),
            pltpu.VMEM((CHUNK, 128), jnp.float32),
            pltpu.VMEM((NEG * CHUNK, 128), jnp.float32),
            pltpu.VMEM((CHUNK,), jnp.float32),
            pltpu.VMEM((NEG * CHUNK,), jnp.float32),
            pltpu.SemaphoreType.DMA,
        ],
    )
    def k(td_hbm, to_hbm, cd_hbm, co_hbm, nd_hbm, no_hbm, iemb_hbm, oemb_hbm,
          pos_hbm, negout_hbm,
          t_idx, t_po, c_idx, c_po, n_idx, n_po,
          t_rows, c_rows, n_rows, pos_v, neg_v, sem):
        wid = lax.axis_index("s") * NC + lax.axis_index("c")
        iota = lax.iota(jnp.int32, L)
        for ch in range(NCHUNK):
            base = wid * B_PER_W + ch * CHUNK
            stage = [
                pltpu.async_copy(td_hbm.at[pl.ds(base, CHUNK)], t_idx, sem),
                pltpu.async_copy(to_hbm.at[pl.ds(base, CHUNK)], t_po, sem),
                pltpu.async_copy(cd_hbm.at[pl.ds(base, CHUNK)], c_idx, sem),
                pltpu.async_copy(co_hbm.at[pl.ds(base, CHUNK)], c_po, sem),
                pltpu.async_copy(
                    nd_hbm.at[pl.ds(base * NEG, NEG * CHUNK)], n_idx, sem),
                pltpu.async_copy(
                    no_hbm.at[pl.ds(base * NEG, NEG * CHUNK)], n_po, sem),
            ]
            for cp in stage:
                cp.wait()
            cps = [
                pltpu.async_copy(iemb_hbm.at[t_idx], t_rows, sem),
                pltpu.async_copy(oemb_hbm.at[c_idx], c_rows, sem),
            ]
            for s in range(NEG):
                cps.append(pltpu.async_copy(
                    oemb_hbm.at[n_idx.at[pl.ds(s * CHUNK, CHUNK)]],
                    n_rows.at[pl.ds(s * CHUNK, CHUNK)], sem))
            for cp in cps:
                cp.wait()

            for blk in range(CHUNK // L):
                rows = blk * L + iota
                n_rowidx = [rows * NEG + kk for kk in range(NEG)]
                t_col0 = t_po[pl.ds(blk * L, L)]
                c_col0 = c_po[pl.ds(blk * L, L)]
                n_col0 = [plsc.load_gather(n_po, [n_rowidx[kk]])
                          for kk in range(NEG)]
                zero = jnp.zeros((L,), jnp.float32)

                def body(dd, carry, rows=rows, n_rowidx=n_rowidx,
                         t_col0=t_col0, c_col0=c_col0, n_col0=n_col0):
                    accp, accn = carry[0], list(carry[1:])
                    tv = plsc.load_gather(t_rows, [rows, t_col0 + dd])
                    cv = plsc.load_gather(c_rows, [rows, c_col0 + dd])
                    accp = accp + tv * cv
                    for kk in range(NEG):
                        nv = plsc.load_gather(
                            n_rows, [n_rowidx[kk], n_col0[kk] + dd])
                        accn[kk] = accn[kk] + tv * nv
                    return (accp, *accn)

                accs = lax.fori_loop(0, DIM, body, (zero,) * (1 + NEG))
                pos_v[pl.ds(blk * L, L)] = accs[0]
                for kk in range(NEG):
                    plsc.store_scatter(neg_v, [n_rowidx[kk]], accs[1 + kk])

            pltpu.sync_copy(pos_v, pos_hbm.at[pl.ds(base, CHUNK)])
            pltpu.sync_copy(neg_v, negout_hbm.at[pl.ds(base * NEG, NEG * CHUNK)])

    return k(t_div, t_off, c_div, c_off, n_div, n_off, ptab_i, ptab_o)


def _tc_loss(pos_scores, neg_scores):
    def body(p_ref, n_ref, pos_out, neg_out):
        p = p_ref[...]
        n = n_ref[...]

        def neg_logsig(x):
            return jnp.log(1.0 + jnp.exp(-jnp.abs(x))) - jnp.minimum(x, 0.0)

        pos_out[0, 0] = jnp.mean(neg_logsig(p))
        neg_out[0, 0] = jnp.mean(neg_logsig(-n))

    o1, o2 = pl.pallas_call(
        body,
        out_shape=[jax.ShapeDtypeStruct((1, 1), jnp.float32)] * 2,
        out_specs=[pl.BlockSpec(memory_space=pltpu.SMEM)] * 2,
    )(pos_scores.reshape(BATCH // 128, 128),
      neg_scores.reshape(BATCH * NEG // 128, 128))
    return o1[0, 0], o2[0, 0]


def kernel(target_words, context_words, negative_words, input_emb, output_emb):
    ptab_i = _repack(input_emb.T)
    ptab_o = _repack(output_emb.T)
    nf = negative_words.reshape(BATCH * NEG)

    shift = RW.bit_length() - 1

    def split(ix):
        blk = jnp.right_shift(ix, shift)
        loc = jnp.bitwise_and(ix, RW - 1)
        div = jnp.left_shift(blk, shift - 1) + jnp.bitwise_and(loc, RW // 2 - 1)
        off = jnp.left_shift(jnp.right_shift(loc, shift - 1), 6)
        return div, off

    t_div, t_off = split(target_words)
    c_div, c_off = split(context_words)
    n_div, n_off = split(nf)
    pos_s, neg_s = _sc_scores(t_div, t_off, c_div, c_off, n_div, n_off,
                              ptab_i, ptab_o)
    return _tc_loss(pos_s, neg_s)


# speedup vs baseline: 5.3836x; 1.1016x over previous
import functools

import jax
import jax.numpy as jnp
from jax import lax
from jax.experimental import pallas as pl
from jax.experimental.pallas import tpu as pltpu
from jax.experimental.pallas import tpu_sc as plsc

VOCAB = 1000000
DIM = 64
BATCH = 16384
NEG = 5

NC = 2
NS = 16
L = 16
NW = NC * NS
B_PER_W = BATCH // NW
CHUNK = 128
NCHUNK = B_PER_W // CHUNK

RW = 16384
NRBLK = -(-VOCAB // RW)
PROWS = NRBLK * (RW // 2)


def _repack(embT):
    def body(in_ref, out_ref):
        a = in_ref[...]
        out_ref[...] = jnp.concatenate(
            [a[:, :RW // 2].T, a[:, RW // 2:].T], axis=1)

    return pl.pallas_call(
        body,
        grid=(NRBLK,),
        in_specs=[pl.BlockSpec((DIM, RW), lambda g: (0, g))],
        out_specs=pl.BlockSpec((RW // 2, 128), lambda g: (g, 0)),
        out_shape=jax.ShapeDtypeStruct((PROWS, 128), jnp.float32),
    )(embT)


def _sc_scores(t_div, t_off, c_div, c_off, n_div, n_off, ptab_i, ptab_o):
    mesh = plsc.VectorSubcoreMesh(core_axis_name="c", subcore_axis_name="s")

    @functools.partial(
        pl.kernel,
        out_type=[
            jax.ShapeDtypeStruct((BATCH,), jnp.float32),
            jax.ShapeDtypeStruct((BATCH * NEG,), jnp.float32),
        ],
        mesh=mesh,
        compiler_params=pltpu.CompilerParams(
            needs_layout_passes=False, use_tc_tiling_on_sc=True),
        scratch_types=[
            pltpu.VMEM((CHUNK,), jnp.int32),
            pltpu.VMEM((CHUNK,), jnp.int32),
            pltpu.VMEM((CHUNK,), jnp.int32),
            pltpu.VMEM((CHUNK,), jnp.int32),
            pltpu.VMEM((NEG * CHUNK,), jnp.int32),
            pltpu.VMEM((NEG * CHUNK,), jnp.int32),
            pltpu.VMEM((CHUNK, 128), jnp.float32),
            pltpu.VMEM((CHUNK, 128), jnp.float32),
            pltpu.VMEM((NEG * CHUNK, 128), jnp.float32),
            pltpu.VMEM((CHUNK,), jnp.float32),
            pltpu.VMEM((NEG * CHUNK,), jnp.float32),
            pltpu.SemaphoreType.DMA,
        ],
    )
    def k(td_hbm, to_hbm, cd_hbm, co_hbm, nd_hbm, no_hbm, iemb_hbm, oemb_hbm,
          pos_hbm, negout_hbm,
          t_idx, t_po, c_idx, c_po, n_idx, n_po,
          t_rows, c_rows, n_rows, pos_v, neg_v, sem):
        wid = lax.axis_index("s") * NC + lax.axis_index("c")
        iota = lax.iota(jnp.int32, L)
        for ch in range(NCHUNK):
            base = wid * B_PER_W + ch * CHUNK
            stage = [
                pltpu.async_copy(td_hbm.at[pl.ds(base, CHUNK)], t_idx, sem),
                pltpu.async_copy(to_hbm.at[pl.ds(base, CHUNK)], t_po, sem),
                pltpu.async_copy(cd_hbm.at[pl.ds(base, CHUNK)], c_idx, sem),
                pltpu.async_copy(co_hbm.at[pl.ds(base, CHUNK)], c_po, sem),
                pltpu.async_copy(
                    nd_hbm.at[pl.ds(base * NEG, NEG * CHUNK)], n_idx, sem),
                pltpu.async_copy(
                    no_hbm.at[pl.ds(base * NEG, NEG * CHUNK)], n_po, sem),
            ]
            for cp in stage:
                cp.wait()
            cps = [
                pltpu.async_copy(iemb_hbm.at[t_idx], t_rows, sem),
                pltpu.async_copy(oemb_hbm.at[c_idx], c_rows, sem),
            ]
            for s in range(NEG):
                cps.append(pltpu.async_copy(
                    oemb_hbm.at[n_idx.at[pl.ds(s * CHUNK, CHUNK)]],
                    n_rows.at[pl.ds(s * CHUNK, CHUNK)], sem))
            for cp in cps:
                cp.wait()

            for blk in range(CHUNK // L):
                rows = blk * L + iota
                n_rowidx = [rows * NEG + kk for kk in range(NEG)]
                t_col0 = t_po[pl.ds(blk * L, L)]
                c_col0 = c_po[pl.ds(blk * L, L)]
                n_col0 = [plsc.load_gather(n_po, [n_rowidx[kk]])
                          for kk in range(NEG)]
                zero = jnp.zeros((L,), jnp.float32)

                def body(dd, carry, rows=rows, n_rowidx=n_rowidx,
                         t_col0=t_col0, c_col0=c_col0, n_col0=n_col0):
                    accp, accn = carry[0], list(carry[1:])
                    tv = plsc.load_gather(t_rows, [rows, t_col0 + dd])
                    cv = plsc.load_gather(c_rows, [rows, c_col0 + dd])
                    accp = accp + tv * cv
                    for kk in range(NEG):
                        nv = plsc.load_gather(
                            n_rows, [n_rowidx[kk], n_col0[kk] + dd])
                        accn[kk] = accn[kk] + tv * nv
                    return (accp, *accn)

                accs = lax.fori_loop(0, DIM, body, (zero,) * (1 + NEG))
                pos_v[pl.ds(blk * L, L)] = accs[0]
                for kk in range(NEG):
                    plsc.store_scatter(neg_v, [n_rowidx[kk]], accs[1 + kk])

            pltpu.sync_copy(pos_v, pos_hbm.at[pl.ds(base, CHUNK)])
            pltpu.sync_copy(neg_v, negout_hbm.at[pl.ds(base * NEG, NEG * CHUNK)])

    return k(t_div, t_off, c_div, c_off, n_div, n_off, ptab_i, ptab_o)


def _tc_loss(pos_scores, neg_scores):
    def body(p_ref, n_ref, pos_out, neg_out):
        p = p_ref[...]
        n = n_ref[...]

        def neg_logsig(x):
            return jnp.log(1.0 + jnp.exp(-jnp.abs(x))) - jnp.minimum(x, 0.0)

        pos_out[0, 0] = jnp.mean(neg_logsig(p))
        neg_out[0, 0] = jnp.mean(neg_logsig(-n))

    o1, o2 = pl.pallas_call(
        body,
        out_shape=[jax.ShapeDtypeStruct((1, 1), jnp.float32)] * 2,
        out_specs=[pl.BlockSpec(memory_space=pltpu.SMEM)] * 2,
    )(pos_scores.reshape(BATCH // 128, 128),
      neg_scores.reshape(BATCH * NEG // 128, 128))
    return o1[0, 0], o2[0, 0]


def kernel(target_words, context_words, negative_words, input_emb, output_emb):
    ptab_i = _repack(input_emb.T)
    ptab_o = _repack(output_emb.T)
    nf = negative_words.reshape(BATCH * NEG)

    shift = RW.bit_length() - 1

    def split(ix):
        blk = jnp.right_shift(ix, shift)
        loc = jnp.bitwise_and(ix, RW - 1)
        div = jnp.left_shift(blk, shift - 1) + jnp.bitwise_and(loc, RW // 2 - 1)
        off = jnp.left_shift(jnp.right_shift(loc, shift - 1), 6)
        return div, off

    t_div, t_off = split(target_words)
    c_div, c_off = split(context_words)
    n_div, n_off = split(nf)
    pos_s, neg_s = _sc_scores(t_div, t_off, c_div, c_off, n_div, n_off,
                              ptab_i, ptab_o)
    return _tc_loss(pos_s, neg_s)


# speedup vs baseline: 5.6384x; 1.0473x over previous
import functools

import jax
import jax.numpy as jnp
from jax import lax
from jax.experimental import pallas as pl
from jax.experimental.pallas import tpu as pltpu
from jax.experimental.pallas import tpu_sc as plsc

VOCAB = 1000000
DIM = 64
BATCH = 16384
NEG = 5

NC = 2
NS = 16
L = 16
NW = NC * NS
B_PER_W = BATCH // NW
CHUNK = 128
NCHUNK = B_PER_W // CHUNK

RW = 32768
NRBLK = -(-VOCAB // RW)
PROWS = NRBLK * (RW // 2)


def _repack(embT):
    def body(in_ref, out_ref):
        a = in_ref[...]
        out_ref[...] = jnp.concatenate(
            [a[:, :RW // 2].T, a[:, RW // 2:].T], axis=1)

    return pl.pallas_call(
        body,
        grid=(NRBLK,),
        in_specs=[pl.BlockSpec((DIM, RW), lambda g: (0, g))],
        out_specs=pl.BlockSpec((RW // 2, 128), lambda g: (g, 0)),
        out_shape=jax.ShapeDtypeStruct((PROWS, 128), jnp.float32),
    )(embT)


def _sc_scores(t_div, t_off, c_div, c_off, n_div, n_off, ptab_i, ptab_o):
    mesh = plsc.VectorSubcoreMesh(core_axis_name="c", subcore_axis_name="s")

    @functools.partial(
        pl.kernel,
        out_type=[
            jax.ShapeDtypeStruct((BATCH,), jnp.float32),
            jax.ShapeDtypeStruct((BATCH * NEG,), jnp.float32),
        ],
        mesh=mesh,
        compiler_params=pltpu.CompilerParams(
            needs_layout_passes=False, use_tc_tiling_on_sc=True),
        scratch_types=[
            pltpu.VMEM((CHUNK,), jnp.int32),
            pltpu.VMEM((CHUNK,), jnp.int32),
            pltpu.VMEM((CHUNK,), jnp.int32),
            pltpu.VMEM((CHUNK,), jnp.int32),
            pltpu.VMEM((NEG * CHUNK,), jnp.int32),
            pltpu.VMEM((NEG * CHUNK,), jnp.int32),
            pltpu.VMEM((CHUNK, 128), jnp.float32),
            pltpu.VMEM((CHUNK, 128), jnp.float32),
            pltpu.VMEM((NEG * CHUNK, 128), jnp.float32),
            pltpu.VMEM((CHUNK,), jnp.float32),
            pltpu.VMEM((NEG * CHUNK,), jnp.float32),
            pltpu.SemaphoreType.DMA,
        ],
    )
    def k(td_hbm, to_hbm, cd_hbm, co_hbm, nd_hbm, no_hbm, iemb_hbm, oemb_hbm,
          pos_hbm, negout_hbm,
          t_idx, t_po, c_idx, c_po, n_idx, n_po,
          t_rows, c_rows, n_rows, pos_v, neg_v, sem):
        wid = lax.axis_index("s") * NC + lax.axis_index("c")
        iota = lax.iota(jnp.int32, L)
        for ch in range(NCHUNK):
            base = wid * B_PER_W + ch * CHUNK
            stage = [
                pltpu.async_copy(td_hbm.at[pl.ds(base, CHUNK)], t_idx, sem),
                pltpu.async_copy(to_hbm.at[pl.ds(base, CHUNK)], t_po, sem),
                pltpu.async_copy(cd_hbm.at[pl.ds(base, CHUNK)], c_idx, sem),
                pltpu.async_copy(co_hbm.at[pl.ds(base, CHUNK)], c_po, sem),
                pltpu.async_copy(
                    nd_hbm.at[pl.ds(base * NEG, NEG * CHUNK)], n_idx, sem),
                pltpu.async_copy(
                    no_hbm.at[pl.ds(base * NEG, NEG * CHUNK)], n_po, sem),
            ]
            for cp in stage:
                cp.wait()
            cps = [
                pltpu.async_copy(iemb_hbm.at[t_idx], t_rows, sem),
                pltpu.async_copy(oemb_hbm.at[c_idx], c_rows, sem),
            ]
            for s in range(NEG):
                cps.append(pltpu.async_copy(
                    oemb_hbm.at[n_idx.at[pl.ds(s * CHUNK, CHUNK)]],
                    n_rows.at[pl.ds(s * CHUNK, CHUNK)], sem))
            for cp in cps:
                cp.wait()

            for blk in range(CHUNK // L):
                rows = blk * L + iota
                n_rowidx = [rows * NEG + kk for kk in range(NEG)]
                t_col0 = t_po[pl.ds(blk * L, L)]
                c_col0 = c_po[pl.ds(blk * L, L)]
                n_col0 = [plsc.load_gather(n_po, [n_rowidx[kk]])
                          for kk in range(NEG)]
                zero = jnp.zeros((L,), jnp.float32)

                def body(dd, carry, rows=rows, n_rowidx=n_rowidx,
                         t_col0=t_col0, c_col0=c_col0, n_col0=n_col0):
                    accp, accn = carry[0], list(carry[1:])
                    tv = plsc.load_gather(t_rows, [rows, t_col0 + dd])
                    cv = plsc.load_gather(c_rows, [rows, c_col0 + dd])
                    accp = accp + tv * cv
                    for kk in range(NEG):
                        nv = plsc.load_gather(
                            n_rows, [n_rowidx[kk], n_col0[kk] + dd])
                        accn[kk] = accn[kk] + tv * nv
                    return (accp, *accn)

                accs = lax.fori_loop(0, DIM, body, (zero,) * (1 + NEG))
                pos_v[pl.ds(blk * L, L)] = accs[0]
                for kk in range(NEG):
                    plsc.store_scatter(neg_v, [n_rowidx[kk]], accs[1 + kk])

            pltpu.sync_copy(pos_v, pos_hbm.at[pl.ds(base, CHUNK)])
            pltpu.sync_copy(neg_v, negout_hbm.at[pl.ds(base * NEG, NEG * CHUNK)])

    return k(t_div, t_off, c_div, c_off, n_div, n_off, ptab_i, ptab_o)


def _tc_loss(pos_scores, neg_scores):
    def body(p_ref, n_ref, pos_out, neg_out):
        p = p_ref[...]
        n = n_ref[...]

        def neg_logsig(x):
            return jnp.log(1.0 + jnp.exp(-jnp.abs(x))) - jnp.minimum(x, 0.0)

        pos_out[0, 0] = jnp.mean(neg_logsig(p))
        neg_out[0, 0] = jnp.mean(neg_logsig(-n))

    o1, o2 = pl.pallas_call(
        body,
        out_shape=[jax.ShapeDtypeStruct((1, 1), jnp.float32)] * 2,
        out_specs=[pl.BlockSpec(memory_space=pltpu.SMEM)] * 2,
    )(pos_scores.reshape(BATCH // 128, 128),
      neg_scores.reshape(BATCH * NEG // 128, 128))
    return o1[0, 0], o2[0, 0]


def kernel(target_words, context_words, negative_words, input_emb, output_emb):
    ptab_i = _repack(input_emb.T)
    ptab_o = _repack(output_emb.T)
    nf = negative_words.reshape(BATCH * NEG)

    shift = RW.bit_length() - 1

    def split(ix):
        blk = jnp.right_shift(ix, shift)
        loc = jnp.bitwise_and(ix, RW - 1)
        div = jnp.left_shift(blk, shift - 1) + jnp.bitwise_and(loc, RW // 2 - 1)
        off = jnp.left_shift(jnp.right_shift(loc, shift - 1), 6)
        return div, off

    t_div, t_off = split(target_words)
    c_div, c_off = split(context_words)
    n_div, n_off = split(nf)
    pos_s, neg_s = _sc_scores(t_div, t_off, c_div, c_off, n_div, n_off,
                              ptab_i, ptab_o)
    return _tc_loss(pos_s, neg_s)


# speedup vs baseline: 5.8466x; 1.0369x over previous
import functools

import jax
import jax.numpy as jnp
from jax import lax
from jax.experimental import pallas as pl
from jax.experimental.pallas import tpu as pltpu
from jax.experimental.pallas import tpu_sc as plsc

VOCAB = 1000000
DIM = 64
BATCH = 16384
NEG = 5

NC = 2
NS = 16
L = 16
NW = NC * NS
B_PER_W = BATCH // NW
CHUNK = 64
NCHUNK = B_PER_W // CHUNK

RW = 32768
NRBLK = -(-VOCAB // RW)
PROWS = NRBLK * (RW // 2)


def _repack(embT):
    def body(in_ref, out_ref):
        a = in_ref[...]
        out_ref[...] = jnp.concatenate(
            [a[:, :RW // 2].T, a[:, RW // 2:].T], axis=1)

    return pl.pallas_call(
        body,
        grid=(NRBLK,),
        in_specs=[pl.BlockSpec((DIM, RW), lambda g: (0, g))],
        out_specs=pl.BlockSpec((RW // 2, 128), lambda g: (g, 0)),
        out_shape=jax.ShapeDtypeStruct((PROWS, 128), jnp.float32),
    )(embT)


def _sc_scores(t_div, t_off, c_div, c_off, n_div, n_off, ptab_i, ptab_o):
    mesh = plsc.VectorSubcoreMesh(core_axis_name="c", subcore_axis_name="s")

    @functools.partial(
        pl.kernel,
        out_type=[
            jax.ShapeDtypeStruct((BATCH,), jnp.float32),
            jax.ShapeDtypeStruct((BATCH * NEG,), jnp.float32),
        ],
        mesh=mesh,
        compiler_params=pltpu.CompilerParams(
            needs_layout_passes=False, use_tc_tiling_on_sc=True),
        scratch_types=[
            pltpu.VMEM((B_PER_W,), jnp.int32),
            pltpu.VMEM((B_PER_W,), jnp.int32),
            pltpu.VMEM((B_PER_W,), jnp.int32),
            pltpu.VMEM((B_PER_W,), jnp.int32),
            pltpu.VMEM((NEG * B_PER_W,), jnp.int32),
            pltpu.VMEM((NEG * B_PER_W,), jnp.int32),
            pltpu.VMEM((2, CHUNK, 128), jnp.float32),
            pltpu.VMEM((2, CHUNK, 128), jnp.float32),
            pltpu.VMEM((2, NEG * CHUNK, 128), jnp.float32),
            pltpu.VMEM((CHUNK,), jnp.float32),
            pltpu.VMEM((NEG * CHUNK,), jnp.float32),
            pltpu.SemaphoreType.DMA,
        ],
    )
    def k(td_hbm, to_hbm, cd_hbm, co_hbm, nd_hbm, no_hbm, iemb_hbm, oemb_hbm,
          pos_hbm, negout_hbm,
          t_idx, t_po, c_idx, c_po, n_idx, n_po,
          t_rows, c_rows, n_rows, pos_v, neg_v, sem):
        wid = lax.axis_index("s") * NC + lax.axis_index("c")
        iota = lax.iota(jnp.int32, L)
        wbase = wid * B_PER_W
        stage = [
            pltpu.async_copy(td_hbm.at[pl.ds(wbase, B_PER_W)], t_idx, sem),
            pltpu.async_copy(to_hbm.at[pl.ds(wbase, B_PER_W)], t_po, sem),
            pltpu.async_copy(cd_hbm.at[pl.ds(wbase, B_PER_W)], c_idx, sem),
            pltpu.async_copy(co_hbm.at[pl.ds(wbase, B_PER_W)], c_po, sem),
            pltpu.async_copy(
                nd_hbm.at[pl.ds(wbase * NEG, NEG * B_PER_W)], n_idx, sem),
            pltpu.async_copy(
                no_hbm.at[pl.ds(wbase * NEG, NEG * B_PER_W)], n_po, sem),
        ]
        for cp in stage:
            cp.wait()

        def issue(ch, s):
            cps = [
                pltpu.async_copy(
                    iemb_hbm.at[t_idx.at[pl.ds(ch * CHUNK, CHUNK)]],
                    t_rows.at[s], sem),
                pltpu.async_copy(
                    oemb_hbm.at[c_idx.at[pl.ds(ch * CHUNK, CHUNK)]],
                    c_rows.at[s], sem),
            ]
            for g in range(NEG):
                cps.append(pltpu.async_copy(
                    oemb_hbm.at[n_idx.at[
                        pl.ds(ch * NEG * CHUNK + g * CHUNK, CHUNK)]],
                    n_rows.at[s].at[pl.ds(g * CHUNK, CHUNK)], sem))
            return cps

        pend = issue(0, 0)
        for ch in range(NCHUNK):
            s = ch % 2
            if ch + 1 < NCHUNK:
                nxt = issue(ch + 1, 1 - s)
            for cp in pend:
                cp.wait()
            if ch + 1 < NCHUNK:
                pend = nxt

            for blk in range(CHUNK // L):
                rows = blk * L + iota
                n_rowidx = [rows * NEG + kk for kk in range(NEG)]
                t_col0 = t_po[pl.ds(ch * CHUNK + blk * L, L)]
                c_col0 = c_po[pl.ds(ch * CHUNK + blk * L, L)]
                n_col0 = [plsc.load_gather(n_po,
                                           [ch * NEG * CHUNK + n_rowidx[kk]])
                          for kk in range(NEG)]
                zero = jnp.zeros((L,), jnp.float32)
                trs, crs, nrs = t_rows.at[s], c_rows.at[s], n_rows.at[s]

                def body(dd, carry, rows=rows, n_rowidx=n_rowidx,
                         t_col0=t_col0, c_col0=c_col0, n_col0=n_col0,
                         trs=trs, crs=crs, nrs=nrs):
                    accp, accn = carry[0], list(carry[1:])
                    tv = plsc.load_gather(trs, [rows, t_col0 + dd])
                    cv = plsc.load_gather(crs, [rows, c_col0 + dd])
                    accp = accp + tv * cv
                    for kk in range(NEG):
                        nv = plsc.load_gather(
                            nrs, [n_rowidx[kk], n_col0[kk] + dd])
                        accn[kk] = accn[kk] + tv * nv
                    return (accp, *accn)

                accs = lax.fori_loop(0, DIM, body, (zero,) * (1 + NEG))
                pos_v[pl.ds(blk * L, L)] = accs[0]
                for kk in range(NEG):
                    plsc.store_scatter(neg_v, [n_rowidx[kk]], accs[1 + kk])

            base = wbase + ch * CHUNK
            pltpu.sync_copy(pos_v, pos_hbm.at[pl.ds(base, CHUNK)])
            pltpu.sync_copy(neg_v, negout_hbm.at[pl.ds(base * NEG, NEG * CHUNK)])

    return k(t_div, t_off, c_div, c_off, n_div, n_off, ptab_i, ptab_o)


def _tc_loss(pos_scores, neg_scores):
    def body(p_ref, n_ref, pos_out, neg_out):
        p = p_ref[...]
        n = n_ref[...]

        def neg_logsig(x):
            return jnp.log(1.0 + jnp.exp(-jnp.abs(x))) - jnp.minimum(x, 0.0)

        pos_out[0, 0] = jnp.mean(neg_logsig(p))
        neg_out[0, 0] = jnp.mean(neg_logsig(-n))

    o1, o2 = pl.pallas_call(
        body,
        out_shape=[jax.ShapeDtypeStruct((1, 1), jnp.float32)] * 2,
        out_specs=[pl.BlockSpec(memory_space=pltpu.SMEM)] * 2,
    )(pos_scores.reshape(BATCH // 128, 128),
      neg_scores.reshape(BATCH * NEG // 128, 128))
    return o1[0, 0], o2[0, 0]


def kernel(target_words, context_words, negative_words, input_emb, output_emb):
    ptab_i = _repack(input_emb.T)
    ptab_o = _repack(output_emb.T)
    nf = negative_words.reshape(BATCH * NEG)

    shift = RW.bit_length() - 1

    def split(ix):
        blk = jnp.right_shift(ix, shift)
        loc = jnp.bitwise_and(ix, RW - 1)
        div = jnp.left_shift(blk, shift - 1) + jnp.bitwise_and(loc, RW // 2 - 1)
        off = jnp.left_shift(jnp.right_shift(loc, shift - 1), 6)
        return div, off

    t_div, t_off = split(target_words)
    c_div, c_off = split(context_words)
    n_div, n_off = split(nf)
    pos_s, neg_s = _sc_scores(t_div, t_off, c_div, c_off, n_div, n_off,
                              ptab_i, ptab_o)
    return _tc_loss(pos_s, neg_s)
